# baseline (speedup 1.0000x reference)
import functools

import jax
import jax.numpy as jnp
import numpy as np
from jax.experimental import pallas as pl
from jax.experimental.pallas import tpu as pltpu


def _pack_matrix() -> np.ndarray:
    g = np.zeros((128, 128), np.float32)
    for j in range(4):
        for e in range(16):
            src = 8 * e + 2 * j
            dst = 32 * j + 2 * e
            g[src, dst] = -1.0
            g[src, dst + 1] = 1.0
    return g


def _cut_kernel(x_ref, lo_ref, hi_ref, inv_ref, g_ref, out_ref):
    xv = x_ref[...]
    lo = lo_ref[...]
    hi = hi_ref[...]
    invb = inv_ref[...] != 0.0
    cb = jnp.logical_and(xv >= lo, xv <= hi)
    c = jnp.where(cb != invb, 1.0, 0.0).astype(jnp.float32)
    d = c * pltpu.roll(c, 127, 2)
    d = d * pltpu.roll(d, 126, 2)
    d = d * pltpu.roll(d, 124, 2)
    lane = jax.lax.broadcasted_iota(jnp.int32, d.shape, 2)
    d = jnp.where(lane % 8 == 0, d, 0.0)
    p = (d[:, 0, :]
         + pltpu.roll(d[:, 1, :], 2, 1)
         + pltpu.roll(d[:, 2, :], 4, 1)
         + pltpu.roll(d[:, 3, :], 6, 1))
    q = jnp.dot(p, g_ref[...], preferred_element_type=jnp.float32)
    evens = jax.lax.broadcasted_iota(jnp.int32, q.shape, 1) % 2 == 0
    out_ref[...] = q + jnp.where(evens, 1.0, 0.0)


@jax.jit
def kernel(x, cut_left, cut_right, case):
    n, f = x.shape
    assert f == 8 and n % 64 == 0
    rows = n // 64
    blk = 512 if rows % 512 == 0 else rows

    inf = jnp.float32(jnp.inf)
    lo = jnp.where(case == 0, -inf,
         jnp.where(case == 3, jnp.nextafter(cut_left, inf), cut_left))
    hi = jnp.where(case == 0, cut_left,
         jnp.where(case == 1, inf,
         jnp.where(case == 2, cut_right, jnp.nextafter(cut_right, -inf))))
    inv = (case == 3).astype(jnp.float32)

    lane_feat = jnp.asarray(np.tile(np.arange(8), 16), dtype=jnp.int32)
    lo_t = lo[lane_feat].reshape(1, 1, 128)
    hi_t = hi[lane_feat].reshape(1, 1, 128)
    inv_t = inv[lane_feat].reshape(1, 1, 128)
    g = jnp.asarray(_pack_matrix())

    x3 = x.reshape(rows, 4, 128)
    res = pl.pallas_call(
        _cut_kernel,
        grid=(rows // blk,),
        in_specs=[
            pl.BlockSpec((blk, 4, 128), lambda i: (i, 0, 0)),
            pl.BlockSpec((1, 1, 128), lambda i: (0, 0, 0)),
            pl.BlockSpec((1, 1, 128), lambda i: (0, 0, 0)),
            pl.BlockSpec((1, 1, 128), lambda i: (0, 0, 0)),
            pl.BlockSpec((128, 128), lambda i: (0, 0)),
        ],
        out_specs=pl.BlockSpec((blk, 128), lambda i: (i, 0)),
        out_shape=jax.ShapeDtypeStruct((rows, 128), jnp.float32),
    )(x3, lo_t, hi_t, inv_t, g)
    return res.reshape(n, 2)


# speedup vs baseline: 32.4836x; 32.4836x over previous
import jax
import jax.numpy as jnp
from jax.experimental import pallas as pl


def _cut_kernel(x_ref, lo_ref, hi_ref, inv_ref, out_ref):
    xv = x_ref[...]
    lo = lo_ref[...]
    hi = hi_ref[...]
    invb = inv_ref[...] != 0.0
    cb = jnp.logical_and(xv >= lo, xv <= hi)
    c = jnp.where(cb != invb, 1.0, 0.0).astype(jnp.float32)
    p = jnp.min(c, axis=0, keepdims=True)
    out_ref[...] = jnp.concatenate([1.0 - p, p], axis=0)


@jax.jit
def kernel(x, cut_left, cut_right, case):
    n, f = x.shape
    assert f == 8
    blk = 32768 if n % 32768 == 0 else n

    inf = jnp.float32(jnp.inf)
    lo = jnp.where(case == 0, -inf,
         jnp.where(case == 3, jnp.nextafter(cut_left, inf), cut_left))
    hi = jnp.where(case == 0, cut_left,
         jnp.where(case == 1, inf,
         jnp.where(case == 2, cut_right, jnp.nextafter(cut_right, -inf))))
    inv = (case == 3).astype(jnp.float32)

    xt = x.T
    res = pl.pallas_call(
        _cut_kernel,
        grid=(n // blk,),
        in_specs=[
            pl.BlockSpec((8, blk), lambda i: (0, i)),
            pl.BlockSpec((8, 1), lambda i: (0, 0)),
            pl.BlockSpec((8, 1), lambda i: (0, 0)),
            pl.BlockSpec((8, 1), lambda i: (0, 0)),
        ],
        out_specs=pl.BlockSpec((2, blk), lambda i: (0, i)),
        out_shape=jax.ShapeDtypeStruct((2, n), jnp.float32),
    )(xt, lo.reshape(8, 1), hi.reshape(8, 1), inv.reshape(8, 1))
    return res.T
